# baseline (speedup 1.0000x reference)
import functools

import jax
import jax.numpy as jnp
import numpy as np
from jax import lax
from jax.experimental import pallas as pl
from jax.experimental.pallas import tpu as pltpu
from jax.experimental.pallas import tpu_sc as plsc

VOCAB = 1000000
D_MODEL = 64
SEQ_LEN = 2048
BATCH = 16

_info = plsc.get_sparse_core_info()
NC, NS, L = _info.num_cores, _info.num_subcores, _info.num_lanes
NW = NC * NS

TOKENS = BATCH * SEQ_LEN
TOK_PER_W = TOKENS // NW
CHUNK = 32
NCH = TOK_PER_W // CHUNK
NBUF = 4
UNROLL = 4


def _sinusoid_pe(d_model: int, seq_len: int) -> np.ndarray:
    pos = np.arange(seq_len, dtype=np.float64)[:, None]
    i = np.arange(d_model, dtype=np.float64)[None, :]
    denom = np.power(10000.0, (np.floor(i / 2.0) * 2.0) / d_model)
    pe = pos / denom
    pe[:, 0::2] = np.sin(pe[:, 0::2])
    pe[:, 1::2] = np.cos(pe[:, 1::2])
    return pe.astype(np.float32)


_PE = _sinusoid_pe(D_MODEL, SEQ_LEN)

_mesh = plsc.VectorSubcoreMesh(core_axis_name="c", subcore_axis_name="s")


@functools.partial(
    pl.kernel,
    mesh=_mesh,
    out_type=jax.ShapeDtypeStruct((TOKENS, D_MODEL), jnp.float32),
    scratch_types=[
        pltpu.VMEM((TOK_PER_W,), jnp.int32),
        pltpu.VMEM((NBUF, CHUNK, D_MODEL), jnp.float32),
        pltpu.VMEM((NBUF, CHUNK, D_MODEL), jnp.float32),
        [pltpu.SemaphoreType.DMA] * NBUF,
        [pltpu.SemaphoreType.DMA] * NBUF,
        [pltpu.SemaphoreType.DMA] * NBUF,
    ],
)
def _emb_lookup(idx_hbm, pe_hbm, table_hbm, out_hbm,
                idx_v, rbuf, pbuf, semg, semp, semw):
    wid = lax.axis_index("s") * NC + lax.axis_index("c")
    base = wid * TOK_PER_W
    p0 = lax.rem(base, SEQ_LEN)

    pltpu.sync_copy(idx_hbm.at[wid], idx_v)

    def fire_chunk(c, cb):
        pltpu.async_copy(pe_hbm.at[pl.ds(p0 + c * CHUNK, CHUNK)],
                         pbuf.at[cb], semp[cb])
        for g in range(CHUNK // L):
            v = idx_v[pl.ds(c * CHUNK + g * L, L)]
            for l in range(L):
                r = v[l]
                pltpu.async_copy(table_hbm.at[r], rbuf.at[cb, g * L + l],
                                 semg[cb])

    def drain_chunk(c, cb):
        for j in range(CHUNK):
            pltpu.make_async_copy(table_hbm.at[0], rbuf.at[cb, 0],
                                  semg[cb]).wait()
        pltpu.make_async_copy(pe_hbm.at[pl.ds(0, CHUNK)], pbuf.at[cb],
                              semp[cb]).wait()

    for c in range(NBUF - 1):
        fire_chunk(c, c)

    def body(g, carry):
        for u in range(UNROLL):
            cb = u
            c = g * UNROLL + u
            drain_chunk(c, cb)
            for j in range(CHUNK):
                for k in range(D_MODEL // L):
                    sl = pl.ds(k * L, L)
                    rbuf[cb, j, sl] = rbuf[cb, j, sl] + pbuf[cb, j, sl]
            pltpu.async_copy(rbuf.at[cb], out_hbm.at[pl.ds(base + c * CHUNK, CHUNK)],
                             semw[cb])
            nc_ = c + NBUF - 1
            nb = (u + NBUF - 1) % NBUF

            @pl.when(nc_ < NCH)
            def _():
                @pl.when(nc_ >= NBUF)
                def _():
                    pltpu.make_async_copy(
                        rbuf.at[nb], out_hbm.at[pl.ds(0, CHUNK)],
                        semw[nb]).wait()

                fire_chunk(nc_, nb)

        return carry

    lax.fori_loop(0, NCH // UNROLL, body, 0)
    for cb in range(NBUF):
        pltpu.make_async_copy(rbuf.at[cb], out_hbm.at[pl.ds(0, CHUNK)],
                              semw[cb]).wait()


def kernel(x, table):
    idx = x.reshape(NW, TOK_PER_W).astype(jnp.int32)
    out = _emb_lookup(idx, jnp.asarray(_PE), table)
    return out.reshape(BATCH, SEQ_LEN, D_MODEL)
